# baseline (speedup 1.0000x reference)
import functools

import jax
import jax.numpy as jnp
from jax import lax
from jax.experimental import pallas as pl
from jax.experimental.pallas import tpu as pltpu
from jax.experimental.pallas import tpu_sc as plsc

B, P, COORD, FEAT, NCLS, K = 128, 512, 3, 16, 10, 7
E = B * P * K
_NP = float(B * P)
_NE = float(B * P * K)
_F32 = jnp.float32

_pallas_call = pl.pallas_call
_HI = lax.Precision.HIGHEST


def _dotT(x, w):
    return lax.dot_general(x, w, (((1,), (1,)), ((), ())), precision=_HI)


def _dotD(x, w):
    return lax.dot_general(x, w, (((1,), (1,)), ((), ())))


def _affine(s, ss, n, g, b):
    m = s / n
    v = ss / n - m * m
    inv = lax.rsqrt(v + 1e-5)
    sc = g * inv
    return sc, b - m * sc


def _mask_col(f):
    return (jnp.sum(jnp.abs(f), axis=1, keepdims=True) != 0.0).astype(_F32)


def _acc(st_ref, b, rows):
    @pl.when(b == 0)
    def _():
        st_ref[...] = jnp.zeros((8, 128), _F32)

    for r, val in rows:
        c = val.shape[1]
        st_ref[r:r + 1, 0:c] = st_ref[r:r + 1, 0:c] + val


def _sums(y):
    return [(0, jnp.sum(y, axis=0, keepdims=True)),
            (1, jnp.sum(y * y, axis=0, keepdims=True))]


def _topk_store(rank, idx_ref, b):
    it = lax.broadcasted_iota(jnp.int32, (P, P), 1)
    cols = []
    r = rank
    for j in range(K + 1):
        m = jnp.max(r, axis=1, keepdims=True)
        am = jnp.min(jnp.where(r == m, it, P), axis=1, keepdims=True)
        cols.append(am)
        if j < K:
            r = jnp.where(it == am, -jnp.inf, r)
    idx_ref[0] = jnp.concatenate(cols, axis=1) + b * P


def _prep_center(fT, st0, bg, bb):
    sc0, sh0 = _affine(st0[0:1, 0:FEAT], st0[1:2, 0:FEAT], _NP, bg, bb)
    mcol = _mask_col(fT)
    fts = (fT * sc0 + sh0) * mcol
    return fts, mcol, sc0, sh0


def _head_y(fts, gn, w0, cin, dot):
    w0a = w0[:, :cin]
    w0b = w0[:, cin:]
    u = dot(fts, w0a - w0b)
    v = dot(gn, w0b)
    return v + jnp.concatenate([u] * K, axis=0)


def _kmean(h, c):
    acc = h[0:P]
    for k in range(1, K):
        acc = acc + h[k * P:(k + 1) * P]
    return acc / float(K)


def _p1_body(feat_ref, featT_ref, pts_ref, ptsT_ref, idx_ref, st_ref):
    b = pl.program_id(0)
    f_cm = feat_ref[0]
    fT = featT_ref[0]
    mrow = (jnp.sum(jnp.abs(f_cm), axis=0, keepdims=True) != 0.0).astype(_F32)
    mcol = _mask_col(fT)
    fm = fT * mcol
    _acc(st_ref, b, _sums(fm))
    pc = ptsT_ref[0] * mcol + (1.0 - mcol) * 1e9
    pr = pts_ref[0] * mrow + (1.0 - mrow) * 1e9
    g = (pc[:, 0:1] * pr[0:1, :] + pc[:, 1:2] * pr[1:2, :]
         + pc[:, 2:3] * pr[2:3, :])
    xxr = jnp.sum(pr * pr, axis=0, keepdims=True)
    xxc = jnp.sum(pc * pc, axis=1, keepdims=True)
    _topk_store((2.0 * g - xxr) - xxc, idx_ref, b)


def _gather_rows(table, eidx, d):
    info = plsc.get_sparse_core_info()
    nw = info.num_cores * info.num_subcores
    ch = 2048
    per_w = E // nw
    nch = per_w // ch
    mesh = plsc.VectorSubcoreMesh(core_axis_name="c", subcore_axis_name="s")

    @functools.partial(
        pl.kernel, mesh=mesh,
        out_type=jax.ShapeDtypeStruct((E, d), _F32),
        compiler_params=pltpu.CompilerParams(use_tc_tiling_on_sc=False),
        scratch_types=[
            pltpu.VMEM((ch,), jnp.int32),
            pltpu.VMEM((ch, d), _F32),
            pltpu.SemaphoreType.DMA,
        ])
    def gk(table_hbm, idx_hbm, out_hbm, idx_v, rows_v, sem):
        wid = lax.axis_index("s") * info.num_cores + lax.axis_index("c")
        base0 = wid * per_w
        for c in range(nch):
            base = base0 + c * ch
            pltpu.sync_copy(idx_hbm.at[pl.ds(base, ch)], idx_v)
            pltpu.async_copy(table_hbm.at[idx_v], rows_v, sem).wait()
            pltpu.sync_copy(rows_v, out_hbm.at[pl.ds(base, ch)])

    return gk(table, eidx)


def _ec1_head_body(fT, g, st0, bg, bb, w0, scw, y_out, st_out):
    b = pl.program_id(0)
    fts, _, sc0, sh0 = _prep_center(fT[0], st0, bg[...], bb[...])
    gr = g[0]
    gn = (gr * sc0 + sh0) * _mask_col(gr)
    y = _head_y(fts, gn, w0[...], FEAT, _dotT)
    y_out[0] = y
    sc1 = _dotT(fts, scw[...])
    _acc(st_out, b, _sums(y) + [(r + 2, v) for r, v in _sums(sc1)])


def _ec2_head_body(o1, g, w0, y_out, st_out):
    b = pl.program_id(0)
    y = _head_y(o1[0], g[0], w0[...], 32, _dotD)
    y_out[0] = y
    _acc(st_out, b, _sums(y))


def _make_mid_body(dot):
    def _mid_body(y_in, stp, gam, bet, w, y_out, st_out):
        b = pl.program_id(0)
        c = w.shape[0]
        a, sh = _affine(stp[0:1, 0:c], stp[1:2, 0:c], _NE, gam[...], bet[...])
        h = jnp.maximum(y_in[0] * a + sh, 0.0)
        y = dot(h, w[...])
        y_out[0] = y
        _acc(st_out, b, _sums(y))
    return _mid_body


def _p6_body(y3_in, fT, st0, bg, bb, stA, stC, g2r, b2r, scw, scg, scb, sc2w,
             out1_ref, idx2_ref, st2_ref):
    b = pl.program_id(0)
    a3, s3 = _affine(stC[0:1, 0:32], stC[1:2, 0:32], _NE, g2r[...], b2r[...])
    h3 = jnp.maximum(y3_in[0] * a3 + s3, 0.0)
    fmean = _kmean(h3, 32)
    fts, mcol, _, _ = _prep_center(fT[0], st0, bg[...], bb[...])
    sc1 = _dotT(fts, scw[...])
    asc, ssc = _affine(stA[2:3, 0:32], stA[3:4, 0:32], _NP, scg[...], scb[...])
    out1 = jnp.maximum(sc1 * asc + ssc + fmean, 0.0) * mcol
    out1_ref[0] = out1
    pts2 = out1 + (1.0 - mcol) * 1e9
    gm = _dotT(pts2, pts2)
    xxr = _dotT(jnp.ones((1, 32), _F32), pts2 * pts2)
    xxc = jnp.sum(pts2 * pts2, axis=1, keepdims=True)
    _topk_store((2.0 * gm - xxr) - xxc, idx2_ref, b)
    sc2 = _dotD(out1, sc2w[...])
    _acc(st2_ref, b, _sums(sc2))


def _p11_body(y3_in, o1, fT, stC, g2r, b2r, stSC, scg, scb, sc2w, fwa, fwb,
              fz_ref, stF_ref):
    b = pl.program_id(0)
    mcol = _mask_col(fT[0])
    out1 = o1[0]
    a3, s3 = _affine(stC[0:1, 0:64], stC[1:2, 0:64], _NE, g2r[...], b2r[...])
    h3 = jnp.maximum(y3_in[0] * a3 + s3, 0.0)
    fmean = _kmean(h3, 64)
    sc2 = _dotD(out1, sc2w[...])
    asc, ssc = _affine(stSC[0:1, 0:64], stSC[1:2, 0:64], _NP, scg[...],
                       scb[...])
    out2 = jnp.maximum(sc2 * asc + ssc + fmean, 0.0) * mcol
    fz = _dotD(out1, fwa[...]) + _dotD(out2, fwb[...])
    fz_ref[0] = fz
    _acc(stF_ref, b, _sums(fz))


def _p12_body(fz_ref, fT_ref, stF, fg, fb, w1, b1, w2, b2, out_ref, xs):
    b = pl.program_id(0)
    a, sh = _affine(stF[0:1, :], stF[1:2, :], _NP, fg[...], fb[...])
    mcol = _mask_col(fT_ref[0])
    fused = jnp.maximum(fz_ref[0] * a + sh, 0.0) * mcol
    cnt = jnp.maximum(jnp.sum(mcol, axis=0, keepdims=True), 1.0)
    xs[pl.ds(b, 1), :] = jnp.sum(fused, axis=0, keepdims=True) / cnt

    @pl.when(b == B - 1)
    def _():
        x = xs[...]
        h = jnp.maximum(_dotT(x, w1[...]) + b1[...], 0.0)
        out_ref[...] = _dotT(h, w2[...]) + b2[...]


def _c(shape):
    return pl.BlockSpec(shape, lambda b: tuple(0 for _ in shape))


def _s(shape):
    return pl.BlockSpec((1,) + shape[1:], lambda b: (b,) + (0,) * (len(shape) - 1))


_S88 = jax.ShapeDtypeStruct((8, 128), _F32)


def _y_shape(c):
    return jax.ShapeDtypeStruct((B, P * K, c), _F32)


def kernel(points, features, bn_fts_g, bn_fts_b, ec1_w0, ec1_w1, ec1_w2,
           ec1_bn0_g, ec1_bn0_b, ec1_bn1_g, ec1_bn1_b, ec1_bn2_g, ec1_bn2_b,
           ec1_sc_w, ec1_sc_g, ec1_sc_b, ec2_w0, ec2_w1, ec2_w2,
           ec2_bn0_g, ec2_bn0_b, ec2_bn1_g, ec2_bn1_b, ec2_bn2_g, ec2_bn2_b,
           ec2_sc_w, ec2_sc_g, ec2_sc_b, fus_w, fus_g, fus_b,
           fc1_w, fc1_b, fc2_w, fc2_b):
    r1 = lambda v: v.reshape(1, -1)
    featT = jnp.transpose(features, (0, 2, 1))
    ptsT = jnp.transpose(points, (0, 2, 1))

    idx1, st0 = _pallas_call(
        _p1_body, grid=(B,),
        in_specs=[_s((B, FEAT, P)), _s((B, P, FEAT)),
                  _s((B, COORD, P)), _s((B, P, COORD))],
        out_specs=[_s((B, P, K + 1)), _c((8, 128))],
        out_shape=[jax.ShapeDtypeStruct((B, P, K + 1), jnp.int32), _S88],
    )(features, featT, points, ptsT)

    eidx1 = idx1[:, :, 1:].transpose(0, 2, 1).reshape(E)
    g1 = _gather_rows(featT.reshape(B * P, FEAT), eidx1, FEAT)
    g1 = g1.reshape(B, P * K, FEAT)

    bg, bb = r1(bn_fts_g), r1(bn_fts_b)
    ec1_bns = [r1(ec1_bn0_g), r1(ec1_bn0_b), r1(ec1_bn1_g), r1(ec1_bn1_b),
               r1(ec1_bn2_g), r1(ec1_bn2_b)]

    y1, stA1 = _pallas_call(
        _ec1_head_body, grid=(B,),
        in_specs=[_s((B, P, FEAT)), _s((B, P * K, FEAT)), _c((8, 128)),
                  _c((1, FEAT)), _c((1, FEAT)), _c((32, 32)), _c((32, FEAT))],
        out_specs=[_s((B, P * K, 32)), _c((8, 128))],
        out_shape=[_y_shape(32), _S88],
    )(featT, g1, st0, bg, bb, ec1_w0, ec1_sc_w)

    def mid(y_in, stp, gam, bet, w, c, dot):
        return _pallas_call(
            _make_mid_body(dot), grid=(B,),
            in_specs=[_s((B, P * K, c)), _c((8, 128)), _c((1, c)),
                      _c((1, c)), _c((c, c))],
            out_specs=[_s((B, P * K, c)), _c((8, 128))],
            out_shape=[_y_shape(c), _S88],
        )(y_in, stp, gam, bet, w)

    y2, stB1 = mid(y1, stA1, ec1_bns[0], ec1_bns[1], ec1_w1, 32, _dotT)
    y3, stC1 = mid(y2, stB1, ec1_bns[2], ec1_bns[3], ec1_w2, 32, _dotT)

    out1T, idx2, stSC2 = _pallas_call(
        _p6_body, grid=(B,),
        in_specs=[_s((B, P * K, 32)), _s((B, P, FEAT)), _c((8, 128)),
                  _c((1, FEAT)), _c((1, FEAT)), _c((8, 128)), _c((8, 128)),
                  _c((1, 32)), _c((1, 32)), _c((32, FEAT)), _c((1, 32)),
                  _c((1, 32)), _c((64, 32))],
        out_specs=[_s((B, P, 32)), _s((B, P, K + 1)), _c((8, 128))],
        out_shape=[jax.ShapeDtypeStruct((B, P, 32), _F32),
                   jax.ShapeDtypeStruct((B, P, K + 1), jnp.int32), _S88],
    )(y3, featT, st0, bg, bb, stA1, stC1, ec1_bns[4], ec1_bns[5],
      ec1_sc_w, r1(ec1_sc_g), r1(ec1_sc_b), ec2_sc_w)

    eidx2 = idx2[:, :, 1:].transpose(0, 2, 1).reshape(E)
    g2 = _gather_rows(out1T.reshape(B * P, 32), eidx2, 32)
    g2 = g2.reshape(B, P * K, 32)

    ec2_bns = [r1(ec2_bn0_g), r1(ec2_bn0_b), r1(ec2_bn1_g), r1(ec2_bn1_b),
               r1(ec2_bn2_g), r1(ec2_bn2_b)]

    y1b, stA2 = _pallas_call(
        _ec2_head_body, grid=(B,),
        in_specs=[_s((B, P, 32)), _s((B, P * K, 32)), _c((64, 64))],
        out_specs=[_s((B, P * K, 64)), _c((8, 128))],
        out_shape=[_y_shape(64), _S88],
    )(out1T, g2, ec2_w0)

    y2b, stB2 = mid(y1b, stA2, ec2_bns[0], ec2_bns[1], ec2_w1, 64, _dotD)
    y3b, stC2 = mid(y2b, stB2, ec2_bns[2], ec2_bns[3], ec2_w2, 64, _dotD)

    fwa = fus_w[:, :32]
    fwb = fus_w[:, 32:]
    fz, stF = _pallas_call(
        _p11_body, grid=(B,),
        in_specs=[_s((B, P * K, 64)), _s((B, P, 32)), _s((B, P, FEAT)),
                  _c((8, 128)), _c((1, 64)), _c((1, 64)), _c((8, 128)),
                  _c((1, 64)), _c((1, 64)), _c((64, 32)), _c((128, 32)),
                  _c((128, 64))],
        out_specs=[_s((B, P, 128)), _c((8, 128))],
        out_shape=[jax.ShapeDtypeStruct((B, P, 128), _F32), _S88],
    )(y3b, out1T, featT, stC2, ec2_bns[4], ec2_bns[5], stSC2,
      r1(ec2_sc_g), r1(ec2_sc_b), ec2_sc_w, fwa, fwb)

    logits = _pallas_call(
        _p12_body, grid=(B,),
        in_specs=[_s((B, P, 128)), _s((B, P, FEAT)), _c((8, 128)),
                  _c((1, 128)), _c((1, 128)), _c((128, 128)), _c((1, 128)),
                  _c((NCLS, 128)), _c((1, NCLS))],
        out_specs=_c((B, NCLS)),
        out_shape=jax.ShapeDtypeStruct((B, NCLS), _F32),
        scratch_shapes=[pltpu.VMEM((B, 128), _F32)],
    )(fz, featT, stF, r1(fus_g), r1(fus_b), fc1_w, r1(fc1_b), fc2_w,
      r1(fc2_b))

    return logits


# speedup vs baseline: 1.1495x; 1.1495x over previous
import functools

import jax
import jax.numpy as jnp
from jax import lax
from jax.experimental import pallas as pl
from jax.experimental.pallas import tpu as pltpu
from jax.experimental.pallas import tpu_sc as plsc

B, P, COORD, FEAT, NCLS, K = 128, 512, 3, 16, 10, 7
E = B * P * K
NS = 4
GB = B // NS
PK = P * K
_NP = float(B * P)
_NE = float(B * P * K)
_F32 = jnp.float32

_pallas_call = functools.partial(
    pl.pallas_call,
    compiler_params=pltpu.CompilerParams(vmem_limit_bytes=100 * 1024 * 1024))
_HI = lax.Precision.HIGHEST


def _dotT(x, w):
    return lax.dot_general(x, w, (((1,), (1,)), ((), ())), precision=_HI)


def _dotD(x, w):
    return lax.dot_general(x, w, (((1,), (1,)), ((), ())))


def _affine(s, ss, n, g, b):
    m = s / n
    v = ss / n - m * m
    inv = lax.rsqrt(v + 1e-5)
    sc = g * inv
    return sc, b - m * sc


def _mask_col(f):
    return (jnp.sum(jnp.abs(f), axis=1, keepdims=True) != 0.0).astype(_F32)


def _acc(st_ref, b, rows):
    @pl.when(b == 0)
    def _():
        st_ref[...] = jnp.zeros((8, 128), _F32)

    for r, val in rows:
        c = val.shape[1]
        st_ref[r:r + 1, 0:c] = st_ref[r:r + 1, 0:c] + val


def _sums(y):
    return [(0, jnp.sum(y, axis=0, keepdims=True)),
            (1, jnp.sum(y * y, axis=0, keepdims=True))]


def _topk_store(rank, idx_ref, s, base):
    it = lax.broadcasted_iota(jnp.int32, (P, P), 1)
    cols = []
    r = rank
    for j in range(K + 1):
        m = jnp.max(r, axis=1, keepdims=True)
        am = jnp.min(jnp.where(r == m, it, P), axis=1, keepdims=True)
        cols.append(am)
        if j < K:
            r = jnp.where(it == am, -jnp.inf, r)
    idx_ref[s] = jnp.concatenate(cols, axis=1) + base


def _prep_center(fT, st0, bg, bb):
    sc0, sh0 = _affine(st0[0:1, 0:FEAT], st0[1:2, 0:FEAT], _NP, bg, bb)
    mcol = _mask_col(fT)
    fts = (fT * sc0 + sh0) * mcol
    return fts, mcol, sc0, sh0


def _head_y(fts, gn, w0, cin, dot):
    w0a = w0[:, :cin]
    w0b = w0[:, cin:]
    u = dot(fts, w0a - w0b)
    v = dot(gn, w0b)
    rep = jnp.concatenate(
        [u[s * P:(s + 1) * P] for s in range(NS) for _ in range(K)], axis=0)
    return v + rep


def _kmean(h):
    outs = []
    for s in range(NS):
        base = s * PK
        acc = h[base:base + P]
        for k in range(1, K):
            acc = acc + h[base + k * P:base + (k + 1) * P]
        outs.append(acc / float(K))
    return jnp.concatenate(outs, axis=0)


def _p1_body(feat_ref, featT_ref, pts_ref, ptsT_ref, idx_ref, st_ref):
    b = pl.program_id(0)
    fT_all = featT_ref[...].reshape(NS * P, FEAT)
    mcol_all = _mask_col(fT_all)
    fm = fT_all * mcol_all
    _acc(st_ref, b, _sums(fm))
    for s in range(NS):
        f_cm = feat_ref[s]
        mrow = (jnp.sum(jnp.abs(f_cm), axis=0, keepdims=True)
                != 0.0).astype(_F32)
        mcol = mcol_all[s * P:(s + 1) * P]
        pc = ptsT_ref[s] * mcol + (1.0 - mcol) * 1e9
        pr = pts_ref[s] * mrow + (1.0 - mrow) * 1e9
        g = (pc[:, 0:1] * pr[0:1, :] + pc[:, 1:2] * pr[1:2, :]
             + pc[:, 2:3] * pr[2:3, :])
        xxr = jnp.sum(pr * pr, axis=0, keepdims=True)
        xxc = jnp.sum(pc * pc, axis=1, keepdims=True)
        _topk_store((2.0 * g - xxr) - xxc, idx_ref, s, (b * NS + s) * P)


def _gather_rows(table, eidx, d):
    info = plsc.get_sparse_core_info()
    nw = info.num_cores * info.num_subcores
    ch = 2048
    per_w = E // nw
    nch = per_w // ch
    mesh = plsc.VectorSubcoreMesh(core_axis_name="c", subcore_axis_name="s")

    @functools.partial(
        pl.kernel, mesh=mesh,
        out_type=jax.ShapeDtypeStruct((E, d), _F32),
        compiler_params=pltpu.CompilerParams(use_tc_tiling_on_sc=False),
        scratch_types=[
            pltpu.VMEM((ch,), jnp.int32),
            pltpu.VMEM((ch, d), _F32),
            pltpu.SemaphoreType.DMA,
        ])
    def gk(table_hbm, idx_hbm, out_hbm, idx_v, rows_v, sem):
        wid = lax.axis_index("s") * info.num_cores + lax.axis_index("c")
        base0 = wid * per_w
        for c in range(nch):
            base = base0 + c * ch
            pltpu.sync_copy(idx_hbm.at[pl.ds(base, ch)], idx_v)
            pltpu.async_copy(table_hbm.at[idx_v], rows_v, sem).wait()
            pltpu.sync_copy(rows_v, out_hbm.at[pl.ds(base, ch)])

    return gk(table, eidx)


def _ec1_head_body(fT, g, st0, bg, bb, w0, scw, y_out, st_out):
    b = pl.program_id(0)
    fts, _, sc0, sh0 = _prep_center(fT[...].reshape(NS * P, FEAT), st0,
                                    bg[...], bb[...])
    gr = g[...].reshape(NS * PK, FEAT)
    gn = (gr * sc0 + sh0) * _mask_col(gr)
    y = _head_y(fts, gn, w0[...], FEAT, _dotT)
    y_out[...] = y.reshape(NS, PK, 32)
    sc1 = _dotT(fts, scw[...])
    _acc(st_out, b, _sums(y) + [(r + 2, v) for r, v in _sums(sc1)])


def _ec2_head_body(o1, g, w0, y_out, st_out):
    b = pl.program_id(0)
    y = _head_y(o1[...].reshape(NS * P, 32), g[...].reshape(NS * PK, 32),
                w0[...], 32, _dotD)
    y_out[...] = y.reshape(NS, PK, 64)
    _acc(st_out, b, _sums(y))


def _make_mid_body(dot):
    def _mid_body(y_in, stp, gam, bet, w, y_out, st_out):
        b = pl.program_id(0)
        c = w.shape[0]
        a, sh = _affine(stp[0:1, 0:c], stp[1:2, 0:c], _NE, gam[...], bet[...])
        h = jnp.maximum(y_in[...].reshape(NS * PK, c) * a + sh, 0.0)
        y = dot(h, w[...])
        y_out[...] = y.reshape(NS, PK, c)
        _acc(st_out, b, _sums(y))
    return _mid_body


def _p6_body(y3_in, fT, st0, bg, bb, stA, stC, g2r, b2r, scw, scg, scb, sc2w,
             out1_ref, idx2_ref, st2_ref):
    b = pl.program_id(0)
    a3, s3 = _affine(stC[0:1, 0:32], stC[1:2, 0:32], _NE, g2r[...], b2r[...])
    h3 = jnp.maximum(y3_in[...].reshape(NS * PK, 32) * a3 + s3, 0.0)
    fmean = _kmean(h3)
    fts, mcol, _, _ = _prep_center(fT[...].reshape(NS * P, FEAT), st0,
                                   bg[...], bb[...])
    sc1 = _dotT(fts, scw[...])
    asc, ssc = _affine(stA[2:3, 0:32], stA[3:4, 0:32], _NP, scg[...], scb[...])
    out1 = jnp.maximum(sc1 * asc + ssc + fmean, 0.0) * mcol
    out1_ref[...] = out1.reshape(NS, P, 32)
    ones = jnp.ones((1, 32), _F32)
    for s in range(NS):
        pts2 = out1[s * P:(s + 1) * P] + (1.0 - mcol[s * P:(s + 1) * P]) * 1e9
        gm = _dotT(pts2, pts2)
        xxr = _dotT(ones, pts2 * pts2)
        xxc = jnp.sum(pts2 * pts2, axis=1, keepdims=True)
        _topk_store((2.0 * gm - xxr) - xxc, idx2_ref, s, (b * NS + s) * P)
    sc2 = _dotD(out1, sc2w[...])
    _acc(st2_ref, b, _sums(sc2))


def _p11_body(y3_in, o1, fT, stC, g2r, b2r, stSC, scg, scb, sc2w, fwa, fwb,
              fz_ref, stF_ref):
    b = pl.program_id(0)
    mcol = _mask_col(fT[...].reshape(NS * P, FEAT))
    out1 = o1[...].reshape(NS * P, 32)
    a3, s3 = _affine(stC[0:1, 0:64], stC[1:2, 0:64], _NE, g2r[...], b2r[...])
    h3 = jnp.maximum(y3_in[...].reshape(NS * PK, 64) * a3 + s3, 0.0)
    fmean = _kmean(h3)
    sc2 = _dotD(out1, sc2w[...])
    asc, ssc = _affine(stSC[0:1, 0:64], stSC[1:2, 0:64], _NP, scg[...],
                       scb[...])
    out2 = jnp.maximum(sc2 * asc + ssc + fmean, 0.0) * mcol
    fz = _dotD(out1, fwa[...]) + _dotD(out2, fwb[...])
    fz_ref[...] = fz.reshape(NS, P, 128)
    _acc(stF_ref, b, _sums(fz))


def _p12_body(fz_ref, fT_ref, stF, fg, fb, w1, b1, w2, b2, out_ref, xs):
    b = pl.program_id(0)
    a, sh = _affine(stF[0:1, :], stF[1:2, :], _NP, fg[...], fb[...])
    mcol = _mask_col(fT_ref[...].reshape(NS * P, FEAT))
    fused = jnp.maximum(fz_ref[...].reshape(NS * P, 128) * a + sh, 0.0) * mcol
    for s in range(NS):
        sl = slice(s * P, (s + 1) * P)
        cnt = jnp.maximum(jnp.sum(mcol[sl], axis=0, keepdims=True), 1.0)
        xs[pl.ds(b * NS + s, 1), :] = (
            jnp.sum(fused[sl], axis=0, keepdims=True) / cnt)

    @pl.when(b == GB - 1)
    def _():
        x = xs[...]
        h = jnp.maximum(_dotT(x, w1[...]) + b1[...], 0.0)
        out_ref[...] = _dotT(h, w2[...]) + b2[...]


def _c(shape):
    return pl.BlockSpec(shape, lambda b: tuple(0 for _ in shape))


def _s(shape):
    return pl.BlockSpec((NS,) + shape[1:],
                        lambda b: (b,) + (0,) * (len(shape) - 1))


_S88 = jax.ShapeDtypeStruct((8, 128), _F32)


def _y_shape(c):
    return jax.ShapeDtypeStruct((B, PK, c), _F32)


def kernel(points, features, bn_fts_g, bn_fts_b, ec1_w0, ec1_w1, ec1_w2,
           ec1_bn0_g, ec1_bn0_b, ec1_bn1_g, ec1_bn1_b, ec1_bn2_g, ec1_bn2_b,
           ec1_sc_w, ec1_sc_g, ec1_sc_b, ec2_w0, ec2_w1, ec2_w2,
           ec2_bn0_g, ec2_bn0_b, ec2_bn1_g, ec2_bn1_b, ec2_bn2_g, ec2_bn2_b,
           ec2_sc_w, ec2_sc_g, ec2_sc_b, fus_w, fus_g, fus_b,
           fc1_w, fc1_b, fc2_w, fc2_b):
    r1 = lambda v: v.reshape(1, -1)
    featT = jnp.transpose(features, (0, 2, 1))
    ptsT = jnp.transpose(points, (0, 2, 1))

    idx1, st0 = _pallas_call(
        _p1_body, grid=(GB,),
        in_specs=[_s((B, FEAT, P)), _s((B, P, FEAT)),
                  _s((B, COORD, P)), _s((B, P, COORD))],
        out_specs=[_s((B, P, K + 1)), _c((8, 128))],
        out_shape=[jax.ShapeDtypeStruct((B, P, K + 1), jnp.int32), _S88],
    )(features, featT, points, ptsT)

    eidx1 = idx1[:, :, 1:].transpose(0, 2, 1).reshape(E)
    g1 = _gather_rows(featT.reshape(B * P, FEAT), eidx1, FEAT)
    g1 = g1.reshape(B, PK, FEAT)

    bg, bb = r1(bn_fts_g), r1(bn_fts_b)
    ec1_bns = [r1(ec1_bn0_g), r1(ec1_bn0_b), r1(ec1_bn1_g), r1(ec1_bn1_b),
               r1(ec1_bn2_g), r1(ec1_bn2_b)]

    y1, stA1 = _pallas_call(
        _ec1_head_body, grid=(GB,),
        in_specs=[_s((B, P, FEAT)), _s((B, PK, FEAT)), _c((8, 128)),
                  _c((1, FEAT)), _c((1, FEAT)), _c((32, 32)), _c((32, FEAT))],
        out_specs=[_s((B, PK, 32)), _c((8, 128))],
        out_shape=[_y_shape(32), _S88],
    )(featT, g1, st0, bg, bb, ec1_w0, ec1_sc_w)

    def mid(y_in, stp, gam, bet, w, c, dot):
        return _pallas_call(
            _make_mid_body(dot), grid=(GB,),
            in_specs=[_s((B, PK, c)), _c((8, 128)), _c((1, c)),
                      _c((1, c)), _c((c, c))],
            out_specs=[_s((B, PK, c)), _c((8, 128))],
            out_shape=[_y_shape(c), _S88],
        )(y_in, stp, gam, bet, w)

    y2, stB1 = mid(y1, stA1, ec1_bns[0], ec1_bns[1], ec1_w1, 32, _dotT)
    y3, stC1 = mid(y2, stB1, ec1_bns[2], ec1_bns[3], ec1_w2, 32, _dotT)

    out1T, idx2, stSC2 = _pallas_call(
        _p6_body, grid=(GB,),
        in_specs=[_s((B, PK, 32)), _s((B, P, FEAT)), _c((8, 128)),
                  _c((1, FEAT)), _c((1, FEAT)), _c((8, 128)), _c((8, 128)),
                  _c((1, 32)), _c((1, 32)), _c((32, FEAT)), _c((1, 32)),
                  _c((1, 32)), _c((64, 32))],
        out_specs=[_s((B, P, 32)), _s((B, P, K + 1)), _c((8, 128))],
        out_shape=[jax.ShapeDtypeStruct((B, P, 32), _F32),
                   jax.ShapeDtypeStruct((B, P, K + 1), jnp.int32), _S88],
    )(y3, featT, st0, bg, bb, stA1, stC1, ec1_bns[4], ec1_bns[5],
      ec1_sc_w, r1(ec1_sc_g), r1(ec1_sc_b), ec2_sc_w)

    eidx2 = idx2[:, :, 1:].transpose(0, 2, 1).reshape(E)
    g2 = _gather_rows(out1T.reshape(B * P, 32), eidx2, 32)
    g2 = g2.reshape(B, PK, 32)

    ec2_bns = [r1(ec2_bn0_g), r1(ec2_bn0_b), r1(ec2_bn1_g), r1(ec2_bn1_b),
               r1(ec2_bn2_g), r1(ec2_bn2_b)]

    y1b, stA2 = _pallas_call(
        _ec2_head_body, grid=(GB,),
        in_specs=[_s((B, P, 32)), _s((B, PK, 32)), _c((64, 64))],
        out_specs=[_s((B, PK, 64)), _c((8, 128))],
        out_shape=[_y_shape(64), _S88],
    )(out1T, g2, ec2_w0)

    y2b, stB2 = mid(y1b, stA2, ec2_bns[0], ec2_bns[1], ec2_w1, 64, _dotD)
    y3b, stC2 = mid(y2b, stB2, ec2_bns[2], ec2_bns[3], ec2_w2, 64, _dotD)

    fwa = fus_w[:, :32]
    fwb = fus_w[:, 32:]
    fz, stF = _pallas_call(
        _p11_body, grid=(GB,),
        in_specs=[_s((B, PK, 64)), _s((B, P, 32)), _s((B, P, FEAT)),
                  _c((8, 128)), _c((1, 64)), _c((1, 64)), _c((8, 128)),
                  _c((1, 64)), _c((1, 64)), _c((64, 32)), _c((128, 32)),
                  _c((128, 64))],
        out_specs=[_s((B, P, 128)), _c((8, 128))],
        out_shape=[jax.ShapeDtypeStruct((B, P, 128), _F32), _S88],
    )(y3b, out1T, featT, stC2, ec2_bns[4], ec2_bns[5], stSC2,
      r1(ec2_sc_g), r1(ec2_sc_b), ec2_sc_w, fwa, fwb)

    logits = _pallas_call(
        _p12_body, grid=(GB,),
        in_specs=[_s((B, P, 128)), _s((B, P, FEAT)), _c((8, 128)),
                  _c((1, 128)), _c((1, 128)), _c((128, 128)), _c((1, 128)),
                  _c((NCLS, 128)), _c((1, NCLS))],
        out_specs=_c((B, NCLS)),
        out_shape=jax.ShapeDtypeStruct((B, NCLS), _F32),
        scratch_shapes=[pltpu.VMEM((B, 128), _F32)],
    )(fz, featT, stF, r1(fus_g), r1(fus_b), fc1_w, r1(fc1_b), fc2_w,
      r1(fc2_b))

    return logits


# speedup vs baseline: 1.2074x; 1.0503x over previous
import functools

import jax
import jax.numpy as jnp
from jax import lax
from jax.experimental import pallas as pl
from jax.experimental.pallas import tpu as pltpu
from jax.experimental.pallas import tpu_sc as plsc

B, P, COORD, FEAT, NCLS, K = 128, 512, 3, 16, 10, 7
E = B * P * K
NS = 4
GB = B // NS
PK = P * K
_NP = float(B * P)
_NE = float(B * P * K)
_F32 = jnp.float32

_pallas_call = functools.partial(
    pl.pallas_call,
    compiler_params=pltpu.CompilerParams(vmem_limit_bytes=100 * 1024 * 1024))
_HI = lax.Precision.HIGHEST


def _dotT(x, w):
    return lax.dot_general(x, w, (((1,), (1,)), ((), ())), precision=_HI)


def _dotD(x, w):
    return lax.dot_general(x, w, (((1,), (1,)), ((), ())))


def _affine(s, ss, n, g, b):
    m = s / n
    v = ss / n - m * m
    inv = lax.rsqrt(v + 1e-5)
    sc = g * inv
    return sc, b - m * sc


def _mask_col(f):
    return (jnp.sum(jnp.abs(f), axis=1, keepdims=True) != 0.0).astype(_F32)


def _acc(st_ref, b, rows):
    @pl.when(b == 0)
    def _():
        st_ref[...] = jnp.zeros((8, 128), _F32)

    for r, val in rows:
        c = val.shape[1]
        st_ref[r:r + 1, 0:c] = st_ref[r:r + 1, 0:c] + val


def _sums(y):
    return [(0, jnp.sum(y, axis=0, keepdims=True)),
            (1, jnp.sum(y * y, axis=0, keepdims=True))]


def _topk_store(rank, idx_ref, s, base):
    it = lax.broadcasted_iota(jnp.int32, (P, P), 1)
    cols = []
    r = rank
    for j in range(K + 1):
        m = jnp.max(r, axis=1, keepdims=True)
        am = jnp.min(jnp.where(r == m, it, P), axis=1, keepdims=True)
        cols.append(am)
        if j < K:
            r = jnp.where(it == am, -jnp.inf, r)
    idx_ref[s] = jnp.concatenate(cols, axis=1) + base


def _prep_center(fT, st0, bg, bb):
    sc0, sh0 = _affine(st0[0:1, 0:FEAT], st0[1:2, 0:FEAT], _NP, bg, bb)
    mcol = _mask_col(fT)
    fts = (fT * sc0 + sh0) * mcol
    return fts, mcol, sc0, sh0


def _head_y(fts, gn, w0, cin, dot):
    w0a = w0[:, :cin]
    w0b = w0[:, cin:]
    u = dot(fts, w0a - w0b)
    v = dot(gn, w0b)
    rep = jnp.concatenate(
        [u[s * P:(s + 1) * P] for s in range(NS) for _ in range(K)], axis=0)
    return v + rep


def _kmean(h):
    outs = []
    for s in range(NS):
        base = s * PK
        acc = h[base:base + P]
        for k in range(1, K):
            acc = acc + h[base + k * P:base + (k + 1) * P]
        outs.append(acc / float(K))
    return jnp.concatenate(outs, axis=0)


def _p1_body(feat_ref, featT_ref, pts_ref, ptsT_ref, idx_ref, st_ref):
    b = pl.program_id(0)
    fT_all = featT_ref[...].reshape(NS * P, FEAT)
    mcol_all = _mask_col(fT_all)
    fm = fT_all * mcol_all
    _acc(st_ref, b, _sums(fm))
    for s in range(NS):
        f_cm = feat_ref[s]
        mrow = (jnp.sum(jnp.abs(f_cm), axis=0, keepdims=True)
                != 0.0).astype(_F32)
        mcol = mcol_all[s * P:(s + 1) * P]
        pc = ptsT_ref[s] * mcol + (1.0 - mcol) * 1e9
        pr = pts_ref[s] * mrow + (1.0 - mrow) * 1e9
        g = (pc[:, 0:1] * pr[0:1, :] + pc[:, 1:2] * pr[1:2, :]
             + pc[:, 2:3] * pr[2:3, :])
        xxr = jnp.sum(pr * pr, axis=0, keepdims=True)
        xxc = jnp.sum(pc * pc, axis=1, keepdims=True)
        _topk_store((2.0 * g - xxr) - xxc, idx_ref, s, (b * NS + s) * P)


def _gather_rows(table, eidx, d):
    info = plsc.get_sparse_core_info()
    nw = info.num_cores * info.num_subcores
    ch = 2048 if d <= 16 else 1024
    per_w = E // nw
    nch = per_w // ch
    mesh = plsc.VectorSubcoreMesh(core_axis_name="c", subcore_axis_name="s")

    @functools.partial(
        pl.kernel, mesh=mesh,
        out_type=jax.ShapeDtypeStruct((E, d), _F32),
        compiler_params=pltpu.CompilerParams(use_tc_tiling_on_sc=False),
        scratch_types=[
            pltpu.VMEM((per_w,), jnp.int32),
            pltpu.VMEM((ch, d), _F32),
            pltpu.VMEM((ch, d), _F32),
            pltpu.SemaphoreType.DMA,
            pltpu.SemaphoreType.DMA,
            pltpu.SemaphoreType.DMA,
            pltpu.SemaphoreType.DMA,
        ])
    def gk(table_hbm, idx_hbm, out_hbm, idx_v, r0, r1, sg0, sg1, sw0, sw1):
        wid = lax.axis_index("s") * info.num_cores + lax.axis_index("c")
        base0 = wid * per_w
        pltpu.sync_copy(idx_hbm.at[pl.ds(base0, per_w)], idx_v)
        rows = [r0, r1]
        sg = [sg0, sg1]
        sw = [sw0, sw1]
        gops = [None, None]
        wops = [None, None]
        for c in range(nch):
            i = c % 2
            if wops[i] is not None:
                wops[i].wait()
            gops[i] = pltpu.async_copy(
                table_hbm.at[idx_v.at[pl.ds(c * ch, ch)]], rows[i], sg[i])
            if c >= 1:
                j = (c - 1) % 2
                gops[j].wait()
                wops[j] = pltpu.async_copy(
                    rows[j], out_hbm.at[pl.ds(base0 + (c - 1) * ch, ch)],
                    sw[j])
        i = (nch - 1) % 2
        gops[i].wait()
        pltpu.async_copy(
            rows[i], out_hbm.at[pl.ds(base0 + (nch - 1) * ch, ch)],
            sw[i]).wait()
        if wops[1 - i] is not None:
            wops[1 - i].wait()

    return gk(table, eidx)


def _ec1_head_body(fT, g, st0, bg, bb, w0, scw, y_out, st_out):
    b = pl.program_id(0)
    fts, _, sc0, sh0 = _prep_center(fT[...].reshape(NS * P, FEAT), st0,
                                    bg[...], bb[...])
    gr = g[...].reshape(NS * PK, FEAT)
    gn = (gr * sc0 + sh0) * _mask_col(gr)
    y = _head_y(fts, gn, w0[...], FEAT, _dotT)
    y_out[...] = y.reshape(NS, PK, 32)
    sc1 = _dotT(fts, scw[...])
    _acc(st_out, b, _sums(y) + [(r + 2, v) for r, v in _sums(sc1)])


def _ec2_chain(o1, g, w0, ws, affs):
    fts = o1[...].reshape(NS * P, 32)
    gn = g[...].reshape(NS * PK, 32)
    y = _head_y(fts, gn, w0[...], 32, _dotD)
    for (a, sh), w in zip(affs, ws):
        y = _dotD(jnp.maximum(y * a + sh, 0.0), w[...])
    return y


def _make_ec2_stats(depth):
    def body(*refs):
        if depth == 1:
            o1, g, w0, st = refs
            ws, prior = [], []
        elif depth == 2:
            o1, g, w0, w1, stA, g0, b0, st = refs
            ws, prior = [w1], [(stA, g0, b0)]
        else:
            o1, g, w0, w1, w2, stA, g0, b0, stB, g1r, b1r, st = refs
            ws, prior = [w1, w2], [(stA, g0, b0), (stB, g1r, b1r)]
        b = pl.program_id(0)
        affs = [_affine(sa[0:1, 0:64], sa[1:2, 0:64], _NE, ga[...], ba[...])
                for (sa, ga, ba) in prior]
        y = _ec2_chain(o1, g, w0, ws, affs)
        _acc(st, b, _sums(y))
    return body


def _make_mid_body(dot):
    def _mid_body(y_in, stp, gam, bet, w, y_out, st_out):
        b = pl.program_id(0)
        c = w.shape[0]
        a, sh = _affine(stp[0:1, 0:c], stp[1:2, 0:c], _NE, gam[...], bet[...])
        h = jnp.maximum(y_in[...].reshape(NS * PK, c) * a + sh, 0.0)
        y = dot(h, w[...])
        y_out[...] = y.reshape(NS, PK, c)
        _acc(st_out, b, _sums(y))
    return _mid_body


def _p6_body(y3_in, fT, st0, bg, bb, stA, stC, g2r, b2r, scw, scg, scb, sc2w,
             out1_ref, idx2_ref, st2_ref):
    b = pl.program_id(0)
    a3, s3 = _affine(stC[0:1, 0:32], stC[1:2, 0:32], _NE, g2r[...], b2r[...])
    h3 = jnp.maximum(y3_in[...].reshape(NS * PK, 32) * a3 + s3, 0.0)
    fmean = _kmean(h3)
    fts, mcol, _, _ = _prep_center(fT[...].reshape(NS * P, FEAT), st0,
                                   bg[...], bb[...])
    sc1 = _dotT(fts, scw[...])
    asc, ssc = _affine(stA[2:3, 0:32], stA[3:4, 0:32], _NP, scg[...], scb[...])
    out1 = jnp.maximum(sc1 * asc + ssc + fmean, 0.0) * mcol
    out1_ref[...] = out1.reshape(NS, P, 32)
    ones = jnp.ones((1, 32), _F32)
    for s in range(NS):
        pts2 = out1[s * P:(s + 1) * P] + (1.0 - mcol[s * P:(s + 1) * P]) * 1e9
        gm = _dotT(pts2, pts2)
        xxr = _dotT(ones, pts2 * pts2)
        xxc = jnp.sum(pts2 * pts2, axis=1, keepdims=True)
        _topk_store((2.0 * gm - xxr) - xxc, idx2_ref, s, (b * NS + s) * P)
    sc2 = _dotD(out1, sc2w[...])
    _acc(st2_ref, b, _sums(sc2))


def _p11_body(o1, g, fT, w0, w1, w2, stA, g0, b0, stB, g1r, b1r,
              stC, g2r, b2r, stSC, scg, scb, sc2w, fwa, fwb,
              fz_ref, stF_ref):
    b = pl.program_id(0)
    mcol = _mask_col(fT[...].reshape(NS * P, FEAT))
    out1 = o1[...].reshape(NS * P, 32)
    affs = [_affine(stA[0:1, 0:64], stA[1:2, 0:64], _NE, g0[...], b0[...]),
            _affine(stB[0:1, 0:64], stB[1:2, 0:64], _NE, g1r[...], b1r[...])]
    y3 = _ec2_chain(o1, g, w0, [w1, w2], affs)
    a3, s3 = _affine(stC[0:1, 0:64], stC[1:2, 0:64], _NE, g2r[...], b2r[...])
    h3 = jnp.maximum(y3 * a3 + s3, 0.0)
    fmean = _kmean(h3)
    sc2 = _dotD(out1, sc2w[...])
    asc, ssc = _affine(stSC[0:1, 0:64], stSC[1:2, 0:64], _NP, scg[...],
                       scb[...])
    out2 = jnp.maximum(sc2 * asc + ssc + fmean, 0.0) * mcol
    fz = _dotD(out1, fwa[...]) + _dotD(out2, fwb[...])
    fz_ref[...] = fz.reshape(NS, P, 128)
    _acc(stF_ref, b, _sums(fz))


def _p12_body(fz_ref, fT_ref, stF, fg, fb, w1, b1, w2, b2, out_ref, xs):
    b = pl.program_id(0)
    a, sh = _affine(stF[0:1, :], stF[1:2, :], _NP, fg[...], fb[...])
    mcol = _mask_col(fT_ref[...].reshape(NS * P, FEAT))
    fused = jnp.maximum(fz_ref[...].reshape(NS * P, 128) * a + sh, 0.0) * mcol
    for s in range(NS):
        sl = slice(s * P, (s + 1) * P)
        cnt = jnp.maximum(jnp.sum(mcol[sl], axis=0, keepdims=True), 1.0)
        xs[pl.ds(b * NS + s, 1), :] = (
            jnp.sum(fused[sl], axis=0, keepdims=True) / cnt)

    @pl.when(b == GB - 1)
    def _():
        x = xs[...]
        h = jnp.maximum(_dotT(x, w1[...]) + b1[...], 0.0)
        out_ref[...] = _dotT(h, w2[...]) + b2[...]


def _c(shape):
    return pl.BlockSpec(shape, lambda b: tuple(0 for _ in shape))


def _s(shape):
    return pl.BlockSpec((NS,) + shape[1:],
                        lambda b: (b,) + (0,) * (len(shape) - 1))


_S88 = jax.ShapeDtypeStruct((8, 128), _F32)


def _y_shape(c):
    return jax.ShapeDtypeStruct((B, PK, c), _F32)


def kernel(points, features, bn_fts_g, bn_fts_b, ec1_w0, ec1_w1, ec1_w2,
           ec1_bn0_g, ec1_bn0_b, ec1_bn1_g, ec1_bn1_b, ec1_bn2_g, ec1_bn2_b,
           ec1_sc_w, ec1_sc_g, ec1_sc_b, ec2_w0, ec2_w1, ec2_w2,
           ec2_bn0_g, ec2_bn0_b, ec2_bn1_g, ec2_bn1_b, ec2_bn2_g, ec2_bn2_b,
           ec2_sc_w, ec2_sc_g, ec2_sc_b, fus_w, fus_g, fus_b,
           fc1_w, fc1_b, fc2_w, fc2_b):
    r1 = lambda v: v.reshape(1, -1)
    featT = jnp.transpose(features, (0, 2, 1))
    ptsT = jnp.transpose(points, (0, 2, 1))

    idx1, st0 = _pallas_call(
        _p1_body, grid=(GB,),
        in_specs=[_s((B, FEAT, P)), _s((B, P, FEAT)),
                  _s((B, COORD, P)), _s((B, P, COORD))],
        out_specs=[_s((B, P, K + 1)), _c((8, 128))],
        out_shape=[jax.ShapeDtypeStruct((B, P, K + 1), jnp.int32), _S88],
    )(features, featT, points, ptsT)

    eidx1 = idx1[:, :, 1:].transpose(0, 2, 1).reshape(E)
    g1 = _gather_rows(featT.reshape(B * P, FEAT), eidx1, FEAT)
    g1 = g1.reshape(B, PK, FEAT)

    bg, bb = r1(bn_fts_g), r1(bn_fts_b)
    ec1_bns = [r1(ec1_bn0_g), r1(ec1_bn0_b), r1(ec1_bn1_g), r1(ec1_bn1_b),
               r1(ec1_bn2_g), r1(ec1_bn2_b)]

    y1, stA1 = _pallas_call(
        _ec1_head_body, grid=(GB,),
        in_specs=[_s((B, P, FEAT)), _s((B, PK, FEAT)), _c((8, 128)),
                  _c((1, FEAT)), _c((1, FEAT)), _c((32, 32)), _c((32, FEAT))],
        out_specs=[_s((B, PK, 32)), _c((8, 128))],
        out_shape=[_y_shape(32), _S88],
    )(featT, g1, st0, bg, bb, ec1_w0, ec1_sc_w)

    def mid(y_in, stp, gam, bet, w, c, dot):
        return _pallas_call(
            _make_mid_body(dot), grid=(GB,),
            in_specs=[_s((B, PK, c)), _c((8, 128)), _c((1, c)),
                      _c((1, c)), _c((c, c))],
            out_specs=[_s((B, PK, c)), _c((8, 128))],
            out_shape=[_y_shape(c), _S88],
        )(y_in, stp, gam, bet, w)

    y2, stB1 = mid(y1, stA1, ec1_bns[0], ec1_bns[1], ec1_w1, 32, _dotT)
    y3, stC1 = mid(y2, stB1, ec1_bns[2], ec1_bns[3], ec1_w2, 32, _dotT)

    out1T, idx2, stSC2 = _pallas_call(
        _p6_body, grid=(GB,),
        in_specs=[_s((B, PK, 32)), _s((B, P, FEAT)), _c((8, 128)),
                  _c((1, FEAT)), _c((1, FEAT)), _c((8, 128)), _c((8, 128)),
                  _c((1, 32)), _c((1, 32)), _c((32, FEAT)), _c((1, 32)),
                  _c((1, 32)), _c((64, 32))],
        out_specs=[_s((B, P, 32)), _s((B, P, K + 1)), _c((8, 128))],
        out_shape=[jax.ShapeDtypeStruct((B, P, 32), _F32),
                   jax.ShapeDtypeStruct((B, P, K + 1), jnp.int32), _S88],
    )(y3, featT, st0, bg, bb, stA1, stC1, ec1_bns[4], ec1_bns[5],
      ec1_sc_w, r1(ec1_sc_g), r1(ec1_sc_b), ec2_sc_w)

    eidx2 = idx2[:, :, 1:].transpose(0, 2, 1).reshape(E)
    g2 = _gather_rows(out1T.reshape(B * P, 32), eidx2, 32)
    g2 = g2.reshape(B, PK, 32)

    ec2_bns = [r1(ec2_bn0_g), r1(ec2_bn0_b), r1(ec2_bn1_g), r1(ec2_bn1_b),
               r1(ec2_bn2_g), r1(ec2_bn2_b)]

    stA2 = _pallas_call(
        _make_ec2_stats(1), grid=(GB,),
        in_specs=[_s((B, P, 32)), _s((B, PK, 32)), _c((64, 64))],
        out_specs=_c((8, 128)), out_shape=_S88,
    )(out1T, g2, ec2_w0)

    stB2 = _pallas_call(
        _make_ec2_stats(2), grid=(GB,),
        in_specs=[_s((B, P, 32)), _s((B, PK, 32)), _c((64, 64)),
                  _c((64, 64)), _c((8, 128)), _c((1, 64)), _c((1, 64))],
        out_specs=_c((8, 128)), out_shape=_S88,
    )(out1T, g2, ec2_w0, ec2_w1, stA2, ec2_bns[0], ec2_bns[1])

    stC2 = _pallas_call(
        _make_ec2_stats(3), grid=(GB,),
        in_specs=[_s((B, P, 32)), _s((B, PK, 32)), _c((64, 64)),
                  _c((64, 64)), _c((64, 64)), _c((8, 128)), _c((1, 64)),
                  _c((1, 64)), _c((8, 128)), _c((1, 64)), _c((1, 64))],
        out_specs=_c((8, 128)), out_shape=_S88,
    )(out1T, g2, ec2_w0, ec2_w1, ec2_w2, stA2, ec2_bns[0], ec2_bns[1],
      stB2, ec2_bns[2], ec2_bns[3])

    fwa = fus_w[:, :32]
    fwb = fus_w[:, 32:]
    fz, stF = _pallas_call(
        _p11_body, grid=(GB,),
        in_specs=[_s((B, P, 32)), _s((B, PK, 32)), _s((B, P, FEAT)),
                  _c((64, 64)), _c((64, 64)), _c((64, 64)), _c((8, 128)),
                  _c((1, 64)), _c((1, 64)), _c((8, 128)), _c((1, 64)),
                  _c((1, 64)), _c((8, 128)), _c((1, 64)), _c((1, 64)),
                  _c((8, 128)), _c((1, 64)), _c((1, 64)), _c((64, 32)),
                  _c((128, 32)), _c((128, 64))],
        out_specs=[_s((B, P, 128)), _c((8, 128))],
        out_shape=[jax.ShapeDtypeStruct((B, P, 128), _F32), _S88],
    )(out1T, g2, featT, ec2_w0, ec2_w1, ec2_w2, stA2, ec2_bns[0],
      ec2_bns[1], stB2, ec2_bns[2], ec2_bns[3], stC2, ec2_bns[4],
      ec2_bns[5], stSC2, r1(ec2_sc_g), r1(ec2_sc_b), ec2_sc_w, fwa, fwb)

    logits = _pallas_call(
        _p12_body, grid=(GB,),
        in_specs=[_s((B, P, 128)), _s((B, P, FEAT)), _c((8, 128)),
                  _c((1, 128)), _c((1, 128)), _c((128, 128)), _c((1, 128)),
                  _c((NCLS, 128)), _c((1, NCLS))],
        out_specs=_c((B, NCLS)),
        out_shape=jax.ShapeDtypeStruct((B, NCLS), _F32),
        scratch_shapes=[pltpu.VMEM((B, 128), _F32)],
    )(fz, featT, stF, r1(fus_g), r1(fus_b), fc1_w, r1(fc1_b), fc2_w,
      r1(fc2_b))

    return logits
